# baseline (speedup 1.0000x reference)
import functools

import jax
import jax.numpy as jnp
from jax import lax
from jax.experimental import pallas as pl
from jax.experimental.pallas import tpu as pltpu

_BLOCK_ROWS = 2000


def _fused_mlp_kernel(types_ref, x_ref, w1_ref, b1_ref, w2_ref, b2_ref,
                      spec_ref, o_ref):
    x = x_ref[...]
    h = jnp.tanh(
        jnp.dot(x, w1_ref[...], preferred_element_type=jnp.float32)
        + b1_ref[...])
    val = h * w2_ref[...] + b2_ref[...]
    mask = types_ref[...] == spec_ref[...]
    o_ref[...] = jnp.sum(jnp.where(mask, val, 0.0), axis=1, keepdims=True)


def kernel(features, batch, atom_types, W1, b1, W2, b2):
    del batch
    N, D = features.shape
    S, _, H = W1.shape
    SH = S * H

    w1cat = jnp.transpose(W1, (1, 0, 2)).reshape(D, SH)
    b1cat = b1.reshape(1, SH)
    w2row = W2.reshape(1, SH)
    species_of_lane = (jnp.arange(SH, dtype=jnp.int32) // H).reshape(1, SH)
    b2row = (b2.reshape(1, S)[:, species_of_lane[0]] / H).astype(jnp.float32)

    nb = _BLOCK_ROWS
    pad = (-N) % nb
    if pad:
        features = jnp.pad(features, ((0, pad), (0, 0)))
        atom_types = jnp.pad(atom_types, (0, pad))
    npad = N + pad
    types2d = atom_types.astype(jnp.int32).reshape(npad, 1)

    grid = (npad // nb,)
    out = pl.pallas_call(
        _fused_mlp_kernel,
        grid=grid,
        in_specs=[
            pl.BlockSpec((nb, 1), lambda i: (i, 0)),
            pl.BlockSpec((nb, D), lambda i: (i, 0)),
            pl.BlockSpec((D, SH), lambda i: (0, 0)),
            pl.BlockSpec((1, SH), lambda i: (0, 0)),
            pl.BlockSpec((1, SH), lambda i: (0, 0)),
            pl.BlockSpec((1, SH), lambda i: (0, 0)),
            pl.BlockSpec((1, SH), lambda i: (0, 0)),
        ],
        out_specs=pl.BlockSpec((nb, 1), lambda i: (i, 0)),
        out_shape=jax.ShapeDtypeStruct((npad, 1), jnp.float32),
        compiler_params=pltpu.CompilerParams(
            dimension_semantics=("arbitrary",)),
    )(types2d, features, w1cat, b1cat, w2row, b2row, species_of_lane)
    return out[:N]


# speedup vs baseline: 1.1515x; 1.1515x over previous
import functools

import jax
import jax.numpy as jnp
from jax import lax
from jax.experimental import pallas as pl
from jax.experimental.pallas import tpu as pltpu

_BLOCK_ROWS = 10000


def _fused_mlp_kernel(types_ref, x_ref, w1_ref, b1_ref, w2_ref, b2_ref,
                      spec_ref, o_ref):
    x = x_ref[...]
    h = jnp.tanh(
        jnp.dot(x, w1_ref[...], preferred_element_type=jnp.float32)
        + b1_ref[...])
    val = h * w2_ref[...] + b2_ref[...]
    mask = types_ref[...] == spec_ref[...]
    o_ref[...] = jnp.sum(jnp.where(mask, val, 0.0), axis=1, keepdims=True)


def kernel(features, batch, atom_types, W1, b1, W2, b2):
    del batch
    N, D = features.shape
    S, _, H = W1.shape
    SH = S * H

    w1cat = jnp.transpose(W1, (1, 0, 2)).reshape(D, SH)
    b1cat = b1.reshape(1, SH)
    w2row = W2.reshape(1, SH)
    species_of_lane = (jnp.arange(SH, dtype=jnp.int32) // H).reshape(1, SH)
    b2row = (b2.reshape(1, S)[:, species_of_lane[0]] / H).astype(jnp.float32)

    nb = _BLOCK_ROWS
    pad = (-N) % nb
    if pad:
        features = jnp.pad(features, ((0, pad), (0, 0)))
        atom_types = jnp.pad(atom_types, (0, pad))
    npad = N + pad
    types2d = atom_types.astype(jnp.int32).reshape(npad, 1)

    grid = (npad // nb,)
    out = pl.pallas_call(
        _fused_mlp_kernel,
        grid=grid,
        in_specs=[
            pl.BlockSpec((nb, 1), lambda i: (i, 0)),
            pl.BlockSpec((nb, D), lambda i: (i, 0)),
            pl.BlockSpec((D, SH), lambda i: (0, 0)),
            pl.BlockSpec((1, SH), lambda i: (0, 0)),
            pl.BlockSpec((1, SH), lambda i: (0, 0)),
            pl.BlockSpec((1, SH), lambda i: (0, 0)),
            pl.BlockSpec((1, SH), lambda i: (0, 0)),
        ],
        out_specs=pl.BlockSpec((nb, 1), lambda i: (i, 0)),
        out_shape=jax.ShapeDtypeStruct((npad, 1), jnp.float32),
        compiler_params=pltpu.CompilerParams(
            dimension_semantics=("arbitrary",)),
    )(types2d, features, w1cat, b1cat, w2row, b2row, species_of_lane)
    return out[:N]


# speedup vs baseline: 2.3272x; 2.0211x over previous
import functools

import jax
import jax.numpy as jnp
from jax import lax
from jax.experimental import pallas as pl
from jax.experimental.pallas import tpu as pltpu

_BLOCK_ROWS = 2000


def _fused_mlp_kernel(types_ref, x_ref, w1_ref, b1_ref, w2_ref, b2_ref,
                      spec_ref, o_ref):
    x = x_ref[...]
    ht = lax.dot_general(w1_ref[...], x, (((0,), (1,)), ((), ())),
                         preferred_element_type=jnp.float32)
    ht = jnp.tanh(ht + b1_ref[...])
    val = ht * w2_ref[...] + b2_ref[...]
    mask = spec_ref[...] == types_ref[0]
    out = jnp.sum(jnp.where(mask, val, 0.0), axis=0, keepdims=True)
    o_ref[...] = out.reshape(1, 1, out.shape[1])


def kernel(features, batch, atom_types, W1, b1, W2, b2):
    del batch
    N, D = features.shape
    S, _, H = W1.shape
    SH = S * H

    w1pack = jnp.transpose(W1, (1, 0, 2)).reshape(D, SH)
    b1col = b1.reshape(SH, 1)
    w2col = W2.reshape(SH, 1)
    spec_col = (jnp.arange(SH, dtype=jnp.int32) // H).reshape(SH, 1)
    b2col = (b2.reshape(S)[spec_col[:, 0]] / H).reshape(SH, 1).astype(jnp.float32)

    nb = _BLOCK_ROWS
    assert N % nb == 0, (N, nb)
    grid_n = N // nb
    types3d = atom_types.astype(jnp.int32).reshape(grid_n, 1, nb)

    out3d = pl.pallas_call(
        _fused_mlp_kernel,
        grid=(grid_n,),
        in_specs=[
            pl.BlockSpec((1, 1, nb), lambda i: (i, 0, 0)),
            pl.BlockSpec((nb, D), lambda i: (i, 0)),
            pl.BlockSpec((D, SH), lambda i: (0, 0)),
            pl.BlockSpec((SH, 1), lambda i: (0, 0)),
            pl.BlockSpec((SH, 1), lambda i: (0, 0)),
            pl.BlockSpec((SH, 1), lambda i: (0, 0)),
            pl.BlockSpec((SH, 1), lambda i: (0, 0)),
        ],
        out_specs=pl.BlockSpec((1, 1, nb), lambda i: (i, 0, 0)),
        out_shape=jax.ShapeDtypeStruct((grid_n, 1, nb), jnp.float32),
        compiler_params=pltpu.CompilerParams(
            dimension_semantics=("arbitrary",)),
    )(types3d, features, w1pack, b1col, w2col, b2col, spec_col)
    return out3d.reshape(N, 1)


# speedup vs baseline: 3.6746x; 1.5790x over previous
import functools

import jax
import jax.numpy as jnp
from jax import lax
from jax.experimental import pallas as pl
from jax.experimental.pallas import tpu as pltpu

_BLOCK_ROWS = 5000


def _fused_mlp_kernel(types_ref, x_ref, w1_ref, b1_ref, g2_ref, b2_ref,
                      o_ref):
    x = x_ref[...].astype(jnp.bfloat16)
    ht = lax.dot_general(w1_ref[...], x, (((0,), (1,)), ((), ())),
                         preferred_element_type=jnp.float32)
    th = jnp.tanh(ht + b1_ref[...]).astype(jnp.bfloat16)
    p = lax.dot_general(g2_ref[...], th, (((0,), (0,)), ((), ())),
                        preferred_element_type=jnp.float32)
    p = p + b2_ref[...]
    t = types_ref[0]
    s_count = p.shape[0]
    out = jnp.where(t == 0, p[0:1, :], 0.0)
    for s in range(1, s_count):
        out = jnp.where(t == s, p[s:s + 1, :], out)
    o_ref[...] = out.reshape(1, 1, out.shape[1])


def kernel(features, batch, atom_types, W1, b1, W2, b2):
    del batch
    N, D = features.shape
    S, _, H = W1.shape
    SH = S * H

    w1pack = jnp.transpose(W1, (1, 0, 2)).reshape(D, SH).astype(jnp.bfloat16)
    b1col = b1.reshape(SH, 1)
    spec = jnp.arange(SH, dtype=jnp.int32) // H
    onehot = (spec[:, None] == jnp.arange(S, dtype=jnp.int32)[None, :])
    g2 = (W2.reshape(SH, 1) * onehot).astype(jnp.bfloat16)
    b2col = b2.reshape(S, 1)

    nb = _BLOCK_ROWS
    assert N % nb == 0, (N, nb)
    grid_n = N // nb
    types3d = atom_types.astype(jnp.int32).reshape(grid_n, 1, nb)

    out3d = pl.pallas_call(
        _fused_mlp_kernel,
        grid=(grid_n,),
        in_specs=[
            pl.BlockSpec((1, 1, nb), lambda i: (i, 0, 0)),
            pl.BlockSpec((nb, D), lambda i: (i, 0)),
            pl.BlockSpec((D, SH), lambda i: (0, 0)),
            pl.BlockSpec((SH, 1), lambda i: (0, 0)),
            pl.BlockSpec((SH, S), lambda i: (0, 0)),
            pl.BlockSpec((S, 1), lambda i: (0, 0)),
        ],
        out_specs=pl.BlockSpec((1, 1, nb), lambda i: (i, 0, 0)),
        out_shape=jax.ShapeDtypeStruct((grid_n, 1, nb), jnp.float32),
        compiler_params=pltpu.CompilerParams(
            dimension_semantics=("arbitrary",)),
    )(types3d, features, w1pack, b1col, g2, b2col)
    return out3d.reshape(N, 1)


# speedup vs baseline: 4.3007x; 1.1704x over previous
import functools

import jax
import jax.numpy as jnp
from jax import lax
from jax.experimental import pallas as pl
from jax.experimental.pallas import tpu as pltpu

_BLOCK_ROWS = 10000


def _fused_mlp_kernel(types_ref, x_ref, w1_ref, b1_ref, g2_ref, b2_ref,
                      o_ref):
    x = x_ref[...].astype(jnp.bfloat16)
    ht = lax.dot_general(w1_ref[...], x, (((0,), (1,)), ((), ())),
                         preferred_element_type=jnp.float32)
    th = jnp.tanh(ht + b1_ref[...]).astype(jnp.bfloat16)
    p = lax.dot_general(g2_ref[...], th, (((0,), (0,)), ((), ())),
                        preferred_element_type=jnp.float32)
    p = p + b2_ref[...]
    t = types_ref[0]
    s_count = p.shape[0]
    out = jnp.where(t == 0, p[0:1, :], 0.0)
    for s in range(1, s_count):
        out = jnp.where(t == s, p[s:s + 1, :], out)
    o_ref[...] = out.reshape(1, 1, out.shape[1])


def kernel(features, batch, atom_types, W1, b1, W2, b2):
    del batch
    N, D = features.shape
    S, _, H = W1.shape
    SH = S * H

    w1pack = jnp.transpose(W1, (1, 0, 2)).reshape(D, SH).astype(jnp.bfloat16)
    b1col = b1.reshape(SH, 1)
    spec = jnp.arange(SH, dtype=jnp.int32) // H
    onehot = (spec[:, None] == jnp.arange(S, dtype=jnp.int32)[None, :])
    g2 = (W2.reshape(SH, 1) * onehot).astype(jnp.bfloat16)
    b2col = b2.reshape(S, 1)

    nb = _BLOCK_ROWS
    assert N % nb == 0, (N, nb)
    grid_n = N // nb
    types3d = atom_types.astype(jnp.int32).reshape(grid_n, 1, nb)

    out3d = pl.pallas_call(
        _fused_mlp_kernel,
        grid=(grid_n,),
        in_specs=[
            pl.BlockSpec((1, 1, nb), lambda i: (i, 0, 0)),
            pl.BlockSpec((nb, D), lambda i: (i, 0)),
            pl.BlockSpec((D, SH), lambda i: (0, 0)),
            pl.BlockSpec((SH, 1), lambda i: (0, 0)),
            pl.BlockSpec((SH, S), lambda i: (0, 0)),
            pl.BlockSpec((S, 1), lambda i: (0, 0)),
        ],
        out_specs=pl.BlockSpec((1, 1, nb), lambda i: (i, 0, 0)),
        out_shape=jax.ShapeDtypeStruct((grid_n, 1, nb), jnp.float32),
        compiler_params=pltpu.CompilerParams(
            dimension_semantics=("arbitrary",)),
    )(types3d, features, w1pack, b1col, g2, b2col)
    return out3d.reshape(N, 1)


# speedup vs baseline: 4.4702x; 1.0394x over previous
import functools

import jax
import jax.numpy as jnp
from jax import lax
from jax.experimental import pallas as pl
from jax.experimental.pallas import tpu as pltpu

_BLOCK_ROWS = 20000


def _fused_mlp_kernel(types_ref, x_ref, w1_ref, b1_ref, g2_ref, b2_ref,
                      o_ref):
    x = x_ref[...].astype(jnp.bfloat16)
    ht = lax.dot_general(w1_ref[...], x, (((0,), (1,)), ((), ())),
                         preferred_element_type=jnp.float32)
    th = jnp.tanh(ht + b1_ref[...]).astype(jnp.bfloat16)
    p = lax.dot_general(g2_ref[...], th, (((0,), (0,)), ((), ())),
                        preferred_element_type=jnp.float32)
    p = p + b2_ref[...]
    t = types_ref[0]
    s_count = p.shape[0]
    out = jnp.where(t == 0, p[0:1, :], 0.0)
    for s in range(1, s_count):
        out = jnp.where(t == s, p[s:s + 1, :], out)
    o_ref[...] = out.reshape(1, 1, out.shape[1])


def kernel(features, batch, atom_types, W1, b1, W2, b2):
    del batch
    N, D = features.shape
    S, _, H = W1.shape
    SH = S * H

    w1pack = jnp.transpose(W1, (1, 0, 2)).reshape(D, SH).astype(jnp.bfloat16)
    b1col = b1.reshape(SH, 1)
    spec = jnp.arange(SH, dtype=jnp.int32) // H
    onehot = (spec[:, None] == jnp.arange(S, dtype=jnp.int32)[None, :])
    g2 = (W2.reshape(SH, 1) * onehot).astype(jnp.bfloat16)
    b2col = b2.reshape(S, 1)

    nb = _BLOCK_ROWS
    assert N % nb == 0, (N, nb)
    grid_n = N // nb
    types3d = atom_types.astype(jnp.int32).reshape(grid_n, 1, nb)

    out3d = pl.pallas_call(
        _fused_mlp_kernel,
        grid=(grid_n,),
        in_specs=[
            pl.BlockSpec((1, 1, nb), lambda i: (i, 0, 0)),
            pl.BlockSpec((nb, D), lambda i: (i, 0)),
            pl.BlockSpec((D, SH), lambda i: (0, 0)),
            pl.BlockSpec((SH, 1), lambda i: (0, 0)),
            pl.BlockSpec((SH, S), lambda i: (0, 0)),
            pl.BlockSpec((S, 1), lambda i: (0, 0)),
        ],
        out_specs=pl.BlockSpec((1, 1, nb), lambda i: (i, 0, 0)),
        out_shape=jax.ShapeDtypeStruct((grid_n, 1, nb), jnp.float32),
        compiler_params=pltpu.CompilerParams(
            dimension_semantics=("arbitrary",)),
    )(types3d, features, w1pack, b1col, g2, b2col)
    return out3d.reshape(N, 1)


# speedup vs baseline: 4.4751x; 1.0011x over previous
import functools

import jax
import jax.numpy as jnp
from jax import lax
from jax.experimental import pallas as pl
from jax.experimental.pallas import tpu as pltpu

_BLOCK_ROWS = 20000


def _fused_mlp_kernel(types_ref, x_ref, w1_ref, b1_ref, g2_ref, b2_ref,
                      o_ref):
    x = x_ref[...].astype(jnp.bfloat16)
    ht = lax.dot_general(w1_ref[...], x, (((0,), (1,)), ((), ())),
                         preferred_element_type=jnp.float32)
    th = jnp.tanh(ht + b1_ref[...]).astype(jnp.bfloat16)
    p = lax.dot_general(g2_ref[...], th, (((0,), (0,)), ((), ())),
                        preferred_element_type=jnp.float32)
    p = p + b2_ref[...]
    t = types_ref[0]
    s_count = p.shape[0]
    out = jnp.where(t == 0, p[0:1, :], 0.0)
    for s in range(1, s_count):
        out = jnp.where(t == s, p[s:s + 1, :], out)
    o_ref[...] = out.reshape(1, 1, out.shape[1])


def kernel(features, batch, atom_types, W1, b1, W2, b2):
    del batch
    N, D = features.shape
    S, _, H = W1.shape
    SH = S * H

    w1pack = jnp.transpose(W1, (1, 0, 2)).reshape(D, SH).astype(jnp.bfloat16)
    b1col = b1.reshape(SH, 1)
    spec = jnp.arange(SH, dtype=jnp.int32) // H
    onehot = (spec[:, None] == jnp.arange(S, dtype=jnp.int32)[None, :])
    g2 = (W2.reshape(SH, 1) * onehot).astype(jnp.bfloat16)
    b2col = b2.reshape(S, 1)

    nb = _BLOCK_ROWS
    assert N % nb == 0, (N, nb)
    grid_n = N // nb
    types3d = atom_types.astype(jnp.int32).reshape(grid_n, 1, nb)

    out3d = pl.pallas_call(
        _fused_mlp_kernel,
        grid=(grid_n,),
        in_specs=[
            pl.BlockSpec((1, 1, nb), lambda i: (i, 0, 0)),
            pl.BlockSpec((nb, D), lambda i: (i, 0)),
            pl.BlockSpec((D, SH), lambda i: (0, 0)),
            pl.BlockSpec((SH, 1), lambda i: (0, 0)),
            pl.BlockSpec((SH, S), lambda i: (0, 0)),
            pl.BlockSpec((S, 1), lambda i: (0, 0)),
        ],
        out_specs=pl.BlockSpec((1, 1, nb), lambda i: (i, 0, 0)),
        out_shape=jax.ShapeDtypeStruct((grid_n, 1, nb), jnp.float32),
        compiler_params=pltpu.CompilerParams(
            dimension_semantics=("parallel",)),
    )(types3d, features, w1pack, b1col, g2, b2col)
    return out3d.reshape(N, 1)
